# Initial kernel scaffold; baseline (speedup 1.0000x reference)
import jax
import jax.numpy as jnp
from jax.experimental import pallas as pl


def kernel(sign, s, t, edge_times_batch, h_s, h_s_times, h_s_mask, emb_table, delta_table):
    raise NotImplementedError("write your pallas kernel here")


# speedup vs baseline: 1.0140x; 1.0140x over previous
import functools

import jax
import jax.numpy as jnp
from jax import lax
from jax.experimental import pallas as pl
from jax.experimental.pallas import tpu as pltpu
from jax.experimental.pallas import tpu_sc as plsc

_N = 100000
_D = 128
_H = 5
_B = 100000

_NC, _NS = 2, 16
_NW = _NC * _NS
_W = 3200
_BPAD = _NW * _W
_C = 64
_NCH = _W // _C

_BLK = 512
_G = _BPAD // _BLK


def _gather_body(emb, dtab, sidx, tidx, hidx, s_out, t_out, h_out, d_out,
                 sidx_v, tidx_v, hidx_v, rows_s, rows_t, rows_h, del_v, gsem):
    wid = lax.axis_index("s") * _NC + lax.axis_index("c")
    base = pl.multiple_of(wid * _W, 8)
    pltpu.sync_copy(sidx.at[pl.ds(base, _W)], sidx_v)
    pltpu.sync_copy(tidx.at[pl.ds(base, _W)], tidx_v)
    pltpu.sync_copy(hidx.at[pl.ds(base * _H, _W * _H)], hidx_v)

    def chunk(g, carry):
        off = pl.multiple_of(g * _C, _C)
        cps = [
            pltpu.async_copy(emb.at[sidx_v.at[pl.ds(off, _C)]], rows_s, gsem),
            pltpu.async_copy(emb.at[tidx_v.at[pl.ds(off, _C)]], rows_t, gsem),
            pltpu.async_copy(dtab.at[sidx_v.at[pl.ds(off, _C)]],
                             del_v.at[pl.ds(off, _C)], gsem),
        ]
        for j in range(_H):
            cps.append(pltpu.async_copy(
                emb.at[hidx_v.at[pl.ds(j * _W + off, _C)]],
                rows_h.at[pl.ds(j * _C, _C)], gsem))
        for cp in cps:
            cp.wait()
        gb = base + off
        pltpu.sync_copy(rows_s, s_out.at[pl.ds(gb, _C)])
        pltpu.sync_copy(rows_t, t_out.at[pl.ds(gb, _C)])
        for j in range(_H):
            pltpu.sync_copy(rows_h.at[pl.ds(j * _C, _C)],
                            h_out.at[pl.ds(j * _BPAD + gb, _C)])
        return carry

    lax.fori_loop(0, _NCH, chunk, 0)
    pltpu.sync_copy(del_v, d_out.at[pl.ds(base, _W)])


def _math_body(s_ref, t_ref, h_ref, delta_ref, edge_ref, hst_ref, mask_ref,
               sign_ref, loss_ref):
    s_e = s_ref[...]
    t_e = t_ref[...]
    d = s_e - t_e
    p_mu = -jnp.sum(d * d, axis=-1)
    alphas = []
    for j in range(_H):
        dh = s_e - h_ref[j]
        alphas.append(-jnp.sum(dh * dh, axis=-1))
    m = alphas[0]
    for j in range(1, _H):
        m = jnp.maximum(m, alphas[j])
    es = [jnp.exp(a - m) for a in alphas]
    z_norm = es[0]
    for j in range(1, _H):
        z_norm = z_norm + es[j]
    edge = edge_ref[0, :]
    delta = delta_ref[0, :]
    acc = jnp.zeros_like(p_mu)
    for j in range(_H):
        d_time = edge - hst_ref[j]
        acc = acc + (es[j] / z_norm) * alphas[j] * jnp.exp(-delta * d_time) * mask_ref[j]
    p_lambda = p_mu + acc
    z = sign_ref[0] * p_lambda
    loss_ref[0, :] = jnp.maximum(-z, 0.0) + jnp.log(1.0 + jnp.exp(-jnp.abs(z)))


_sc_mesh = plsc.VectorSubcoreMesh(core_axis_name="c", subcore_axis_name="s")

_gather = pl.kernel(
    _gather_body,
    out_type=(
        jax.ShapeDtypeStruct((_BPAD, _D), jnp.float32),
        jax.ShapeDtypeStruct((_BPAD, _D), jnp.float32),
        jax.ShapeDtypeStruct((_H * _BPAD, _D), jnp.float32),
        jax.ShapeDtypeStruct((_BPAD,), jnp.float32),
    ),
    mesh=_sc_mesh,
    scratch_types=[
        pltpu.VMEM((_W,), jnp.int32),
        pltpu.VMEM((_W,), jnp.int32),
        pltpu.VMEM((_H * _W,), jnp.int32),
        pltpu.VMEM((_C, _D), jnp.float32),
        pltpu.VMEM((_C, _D), jnp.float32),
        pltpu.VMEM((_H * _C, _D), jnp.float32),
        pltpu.VMEM((_W,), jnp.float32),
        pltpu.SemaphoreType.DMA,
    ],
)

_math = pl.pallas_call(
    _math_body,
    grid=(_G,),
    in_specs=[
        pl.BlockSpec((_BLK, _D), lambda i: (i, 0)),
        pl.BlockSpec((_BLK, _D), lambda i: (i, 0)),
        pl.BlockSpec((_H, _BLK, _D), lambda i: (0, i, 0)),
        pl.BlockSpec((1, _BLK), lambda i: (0, i)),
        pl.BlockSpec((1, _BLK), lambda i: (0, i)),
        pl.BlockSpec((_H, _BLK), lambda i: (0, i)),
        pl.BlockSpec((_H, _BLK), lambda i: (0, i)),
        pl.BlockSpec(memory_space=pltpu.SMEM),
    ],
    out_specs=pl.BlockSpec((1, _BLK), lambda i: (0, i)),
    out_shape=jax.ShapeDtypeStruct((1, _BPAD), jnp.float32),
)


def kernel(sign, s, t, edge_times_batch, h_s, h_s_times, h_s_mask, emb_table,
           delta_table):
    def pad(x):
        return jnp.concatenate(
            [x, jnp.zeros((_BPAD - _B,) + x.shape[1:], x.dtype)], axis=0)

    s32 = pad(s.astype(jnp.int32))
    t32 = pad(t.astype(jnp.int32))
    h32 = pad(h_s.astype(jnp.int32))
    h_wjw = h32.reshape(_NW, _W, _H).transpose(0, 2, 1).reshape(-1)
    hstT = pad(h_s_times).T
    maskT = pad(h_s_mask).T
    edge2 = pad(edge_times_batch).reshape(1, _BPAD)

    s_rows, t_rows, h_rows, delta_g = _gather(
        emb_table, delta_table.reshape(_N), s32, t32, h_wjw)

    loss = _math(s_rows, t_rows, h_rows.reshape(_H, _BPAD, _D),
                 delta_g.reshape(1, _BPAD), edge2, hstT, maskT, sign)
    return loss[0, :_B]


# speedup vs baseline: 1.2466x; 1.2294x over previous
import functools

import jax
import jax.numpy as jnp
from jax import lax
from jax.experimental import pallas as pl
from jax.experimental.pallas import tpu as pltpu
from jax.experimental.pallas import tpu_sc as plsc

_N = 100000
_D = 128
_H = 5
_B = 100000

_NC, _NS = 2, 16
_NW = _NC * _NS
_W = 3200
_BPAD = _NW * _W
_C = 40
_NCH = _W // _C

_BLK = 512
_G = _BPAD // _BLK


def _gather_body(emb, dtab, sidx, tidx, hidx, s_out, t_out, h_out, d_out,
                 sidx_v, tidx_v, hidx_v, del_v,
                 rs0, rt0, rh0, rs1, rt1, rh1,
                 gsem0, gsem1, wsem0, wsem1):
    wid = lax.axis_index("s") * _NC + lax.axis_index("c")
    base = pl.multiple_of(wid * _W, 8)
    pltpu.sync_copy(sidx.at[pl.ds(base, _W)], sidx_v)
    pltpu.sync_copy(tidx.at[pl.ds(base, _W)], tidx_v)
    pltpu.sync_copy(hidx.at[pl.ds(base * _H, _W * _H)], hidx_v)

    sets = ((rs0, rt0, rh0, gsem0, wsem0), (rs1, rt1, rh1, gsem1, wsem1))

    def fire_gathers(g, bs):
        rs, rt, rh, gsem, _ = bs
        off = pl.multiple_of(g * _C, 8)
        pltpu.async_copy(emb.at[sidx_v.at[pl.ds(off, _C)]], rs, gsem)
        pltpu.async_copy(emb.at[tidx_v.at[pl.ds(off, _C)]], rt, gsem)
        pltpu.async_copy(dtab.at[sidx_v.at[pl.ds(off, _C)]],
                         del_v.at[pl.ds(off, _C)], gsem)
        for j in range(_H):
            pltpu.async_copy(emb.at[hidx_v.at[pl.ds(j * _W + off, _C)]],
                             rh.at[pl.ds(j * _C, _C)], gsem)

    def drain_gathers(bs):
        rs, rt, rh, gsem, _ = bs
        pltpu.make_async_copy(emb.at[sidx_v.at[pl.ds(0, _C)]], rs, gsem).wait()
        pltpu.make_async_copy(emb.at[sidx_v.at[pl.ds(0, _C)]], rt, gsem).wait()
        pltpu.make_async_copy(dtab.at[sidx_v.at[pl.ds(0, _C)]],
                              del_v.at[pl.ds(0, _C)], gsem).wait()
        for j in range(_H):
            pltpu.make_async_copy(emb.at[sidx_v.at[pl.ds(0, _C)]],
                                  rh.at[pl.ds(j * _C, _C)], gsem).wait()

    def fire_writes(g, bs):
        rs, rt, rh, _, wsem = bs
        gb = base + pl.multiple_of(g * _C, 8)
        pltpu.async_copy(rs, s_out.at[pl.ds(gb, _C)], wsem)
        pltpu.async_copy(rt, t_out.at[pl.ds(gb, _C)], wsem)
        for j in range(_H):
            pltpu.async_copy(rh.at[pl.ds(j * _C, _C)],
                             h_out.at[pl.ds(j * _BPAD + gb, _C)], wsem)

    def drain_writes(bs):
        rs, rt, rh, _, wsem = bs
        pltpu.make_async_copy(rs, s_out.at[pl.ds(base, _C)], wsem).wait()
        pltpu.make_async_copy(rt, t_out.at[pl.ds(base, _C)], wsem).wait()
        for j in range(_H):
            pltpu.make_async_copy(rh.at[pl.ds(j * _C, _C)],
                                  h_out.at[pl.ds(base, _C)], wsem).wait()

    fire_gathers(0, sets[0])

    def outer(i, carry):
        for b in range(2):
            g = 2 * i + b

            @pl.when(g + 1 < _NCH)
            def _fire_next():
                @pl.when(g >= 1)
                def _dw():
                    drain_writes(sets[1 - b])
                fire_gathers(g + 1, sets[1 - b])

            drain_gathers(sets[b])
            fire_writes(g, sets[b])
        return carry

    lax.fori_loop(0, _NCH // 2, outer, 0)
    drain_writes(sets[0])
    drain_writes(sets[1])
    pltpu.sync_copy(del_v, d_out.at[pl.ds(base, _W)])


def _math_body(s_ref, t_ref, h_ref, delta_ref, edge_ref, hst_ref, mask_ref,
               sign_ref, loss_ref):
    s_e = s_ref[...]
    t_e = t_ref[...]
    d = s_e - t_e
    p_mu = -jnp.sum(d * d, axis=-1)
    alphas = []
    for j in range(_H):
        dh = s_e - h_ref[j]
        alphas.append(-jnp.sum(dh * dh, axis=-1))
    m = alphas[0]
    for j in range(1, _H):
        m = jnp.maximum(m, alphas[j])
    es = [jnp.exp(a - m) for a in alphas]
    z_norm = es[0]
    for j in range(1, _H):
        z_norm = z_norm + es[j]
    edge = edge_ref[0, :]
    delta = delta_ref[0, :]
    acc = jnp.zeros_like(p_mu)
    for j in range(_H):
        d_time = edge - hst_ref[j]
        acc = acc + (es[j] / z_norm) * alphas[j] * jnp.exp(-delta * d_time) * mask_ref[j]
    p_lambda = p_mu + acc
    z = sign_ref[0] * p_lambda
    loss_ref[0, :] = jnp.maximum(-z, 0.0) + jnp.log(1.0 + jnp.exp(-jnp.abs(z)))


_sc_mesh = plsc.VectorSubcoreMesh(core_axis_name="c", subcore_axis_name="s")

_gather = pl.kernel(
    _gather_body,
    out_type=(
        jax.ShapeDtypeStruct((_BPAD, _D), jnp.float32),
        jax.ShapeDtypeStruct((_BPAD, _D), jnp.float32),
        jax.ShapeDtypeStruct((_H * _BPAD, _D), jnp.float32),
        jax.ShapeDtypeStruct((_BPAD,), jnp.float32),
    ),
    mesh=_sc_mesh,
    scratch_types=[
        pltpu.VMEM((_W,), jnp.int32),
        pltpu.VMEM((_W,), jnp.int32),
        pltpu.VMEM((_H * _W,), jnp.int32),
        pltpu.VMEM((_W,), jnp.float32),
        pltpu.VMEM((_C, _D), jnp.float32),
        pltpu.VMEM((_C, _D), jnp.float32),
        pltpu.VMEM((_H * _C, _D), jnp.float32),
        pltpu.VMEM((_C, _D), jnp.float32),
        pltpu.VMEM((_C, _D), jnp.float32),
        pltpu.VMEM((_H * _C, _D), jnp.float32),
        pltpu.SemaphoreType.DMA,
        pltpu.SemaphoreType.DMA,
        pltpu.SemaphoreType.DMA,
        pltpu.SemaphoreType.DMA,
    ],
)

_math = pl.pallas_call(
    _math_body,
    grid=(_G,),
    in_specs=[
        pl.BlockSpec((_BLK, _D), lambda i: (i, 0)),
        pl.BlockSpec((_BLK, _D), lambda i: (i, 0)),
        pl.BlockSpec((_H, _BLK, _D), lambda i: (0, i, 0)),
        pl.BlockSpec((1, _BLK), lambda i: (0, i)),
        pl.BlockSpec((1, _BLK), lambda i: (0, i)),
        pl.BlockSpec((_H, _BLK), lambda i: (0, i)),
        pl.BlockSpec((_H, _BLK), lambda i: (0, i)),
        pl.BlockSpec(memory_space=pltpu.SMEM),
    ],
    out_specs=pl.BlockSpec((1, _BLK), lambda i: (0, i)),
    out_shape=jax.ShapeDtypeStruct((1, _BPAD), jnp.float32),
)


def kernel(sign, s, t, edge_times_batch, h_s, h_s_times, h_s_mask, emb_table,
           delta_table):
    def pad(x):
        return jnp.concatenate(
            [x, jnp.zeros((_BPAD - _B,) + x.shape[1:], x.dtype)], axis=0)

    s32 = pad(s.astype(jnp.int32))
    t32 = pad(t.astype(jnp.int32))
    h32 = pad(h_s.astype(jnp.int32))
    h_wjw = h32.reshape(_NW, _W, _H).transpose(0, 2, 1).reshape(-1)
    hstT = pad(h_s_times).T
    maskT = pad(h_s_mask).T
    edge2 = pad(edge_times_batch).reshape(1, _BPAD)

    s_rows, t_rows, h_rows, delta_g = _gather(
        emb_table, delta_table.reshape(_N), s32, t32, h_wjw)

    loss = _math(s_rows, t_rows, h_rows.reshape(_H, _BPAD, _D),
                 delta_g.reshape(1, _BPAD), edge2, hstT, maskT, sign)
    return loss[0, :_B]


# speedup vs baseline: 2.8956x; 2.3229x over previous
import functools

import jax
import jax.numpy as jnp
from jax import lax
from jax.experimental import pallas as pl
from jax.experimental.pallas import tpu as pltpu
from jax.experimental.pallas import tpu_sc as plsc

_N = 100000
_D = 128
_H = 5
_B = 100000

_NC, _NS = 2, 16
_NW = _NC * _NS
_W = 3200
_BPAD = _NW * _W
_C = 40
_NCH = _W // _C

_BLK = 512
_G = _BPAD // _BLK


def _gather_body(emb, dtab, sidx, tidx, hidx, s_out, t_out, h_out, d_out,
                 sidx_v, tidx_v, hidx_v, del_v,
                 rs0, rt0, rh0, rs1, rt1, rh1,
                 gsem0, gsem1, wsem0, wsem1):
    wid = lax.axis_index("s") * _NC + lax.axis_index("c")
    base = pl.multiple_of(wid * _W, 8)
    pltpu.sync_copy(sidx.at[pl.ds(base, _W)], sidx_v)
    pltpu.sync_copy(tidx.at[pl.ds(base, _W)], tidx_v)
    pltpu.sync_copy(hidx.at[pl.ds(base * _H, _W * _H)], hidx_v)

    sets = ((rs0, rt0, rh0, gsem0, wsem0), (rs1, rt1, rh1, gsem1, wsem1))

    def fire_gathers(g, bs):
        rs, rt, rh, gsem, _ = bs
        off = pl.multiple_of(g * _C, 8)
        pltpu.async_copy(emb.at[sidx_v.at[pl.ds(off, _C)]], rs, gsem)
        pltpu.async_copy(emb.at[tidx_v.at[pl.ds(off, _C)]], rt, gsem)
        pltpu.async_copy(dtab.at[sidx_v.at[pl.ds(off, _C)]],
                         del_v.at[pl.ds(off, _C)], gsem)
        for j in range(_H):
            pltpu.async_copy(emb.at[hidx_v.at[pl.ds(j * _W + off, _C)]],
                             rh.at[pl.ds(j * _C, _C)], gsem)

    def drain_gathers(bs):
        rs, rt, rh, gsem, _ = bs
        pltpu.make_async_copy(emb.at[sidx_v.at[pl.ds(0, _C)]], rs, gsem).wait()
        pltpu.make_async_copy(emb.at[sidx_v.at[pl.ds(0, _C)]], rt, gsem).wait()
        pltpu.make_async_copy(dtab.at[sidx_v.at[pl.ds(0, _C)]],
                              del_v.at[pl.ds(0, _C)], gsem).wait()
        for j in range(_H):
            pltpu.make_async_copy(emb.at[sidx_v.at[pl.ds(0, _C)]],
                                  rh.at[pl.ds(j * _C, _C)], gsem).wait()

    def fire_writes(g, bs):
        rs, rt, rh, _, wsem = bs
        gb = base + pl.multiple_of(g * _C, 8)
        pltpu.async_copy(rs, s_out.at[pl.ds(gb, _C)], wsem)
        pltpu.async_copy(rt, t_out.at[pl.ds(gb, _C)], wsem)
        for j in range(_H):
            pltpu.async_copy(rh.at[pl.ds(j * _C, _C)],
                             h_out.at[pl.ds(j * _BPAD + gb, _C)], wsem)

    def drain_writes(bs):
        rs, rt, rh, _, wsem = bs
        pltpu.make_async_copy(rs, s_out.at[pl.ds(base, _C)], wsem).wait()
        pltpu.make_async_copy(rt, t_out.at[pl.ds(base, _C)], wsem).wait()
        for j in range(_H):
            pltpu.make_async_copy(rh.at[pl.ds(j * _C, _C)],
                                  h_out.at[pl.ds(base, _C)], wsem).wait()

    fire_gathers(0, sets[0])

    def outer(i, carry):
        for b in range(2):
            g = 2 * i + b

            @pl.when(g + 1 < _NCH)
            def _fire_next():
                @pl.when(g >= 1)
                def _dw():
                    drain_writes(sets[1 - b])
                fire_gathers(g + 1, sets[1 - b])

            drain_gathers(sets[b])
            fire_writes(g, sets[b])
        return carry

    lax.fori_loop(0, _NCH // 2, outer, 0)
    drain_writes(sets[0])
    drain_writes(sets[1])
    pltpu.sync_copy(del_v, d_out.at[pl.ds(base, _W)])


def _math_body(s_ref, t_ref, h_ref, delta_ref, edge_ref, hst_ref, mask_ref,
               sign_ref, loss_ref):
    s_e = s_ref[...]
    t_e = t_ref[...]
    d = s_e - t_e
    p_mu = -jnp.sum(d * d, axis=-1)
    alphas = []
    for j in range(_H):
        dh = s_e - h_ref[j]
        alphas.append(-jnp.sum(dh * dh, axis=-1))
    m = alphas[0]
    for j in range(1, _H):
        m = jnp.maximum(m, alphas[j])
    es = [jnp.exp(a - m) for a in alphas]
    z_norm = es[0]
    for j in range(1, _H):
        z_norm = z_norm + es[j]
    edge = edge_ref[0, :]
    delta = delta_ref[0, :]
    acc = jnp.zeros_like(p_mu)
    for j in range(_H):
        d_time = edge - hst_ref[j]
        acc = acc + (es[j] / z_norm) * alphas[j] * jnp.exp(-delta * d_time) * mask_ref[j]
    p_lambda = p_mu + acc
    z = sign_ref[0] * p_lambda
    loss_ref[0, :] = jnp.maximum(-z, 0.0) + jnp.log(1.0 + jnp.exp(-jnp.abs(z)))


_sc_mesh = plsc.VectorSubcoreMesh(core_axis_name="c", subcore_axis_name="s")

_gather = pl.kernel(
    _gather_body,
    out_type=(
        jax.ShapeDtypeStruct((_BPAD, _D), jnp.float32),
        jax.ShapeDtypeStruct((_BPAD, _D), jnp.float32),
        jax.ShapeDtypeStruct((_H * _BPAD, _D), jnp.float32),
        jax.ShapeDtypeStruct((_BPAD,), jnp.float32),
    ),
    mesh=_sc_mesh,
    scratch_types=[
        pltpu.VMEM((_W,), jnp.int32),
        pltpu.VMEM((_W,), jnp.int32),
        pltpu.VMEM((_H * _W,), jnp.int32),
        pltpu.VMEM((_W,), jnp.float32),
        pltpu.VMEM((_C, _D), jnp.float32),
        pltpu.VMEM((_C, _D), jnp.float32),
        pltpu.VMEM((_H * _C, _D), jnp.float32),
        pltpu.VMEM((_C, _D), jnp.float32),
        pltpu.VMEM((_C, _D), jnp.float32),
        pltpu.VMEM((_H * _C, _D), jnp.float32),
        pltpu.SemaphoreType.DMA,
        pltpu.SemaphoreType.DMA,
        pltpu.SemaphoreType.DMA,
        pltpu.SemaphoreType.DMA,
    ],
)

_math = pl.pallas_call(
    _math_body,
    grid=(_G,),
    in_specs=[
        pl.BlockSpec((_BLK, _D), lambda i: (i, 0)),
        pl.BlockSpec((_BLK, _D), lambda i: (i, 0)),
        pl.BlockSpec((_H, _BLK, _D), lambda i: (0, i, 0)),
        pl.BlockSpec((1, _BLK), lambda i: (0, i)),
        pl.BlockSpec((1, _BLK), lambda i: (0, i)),
        pl.BlockSpec((_H, _BLK), lambda i: (0, i)),
        pl.BlockSpec((_H, _BLK), lambda i: (0, i)),
        pl.BlockSpec(memory_space=pltpu.SMEM),
    ],
    out_specs=pl.BlockSpec((1, _BLK), lambda i: (0, i)),
    out_shape=jax.ShapeDtypeStruct((1, _BPAD), jnp.float32),
)


def kernel(sign, s, t, edge_times_batch, h_s, h_s_times, h_s_mask, emb_table,
           delta_table):
    def pad(x):
        return jnp.concatenate(
            [x, jnp.zeros((_BPAD - _B,) + x.shape[1:], x.dtype)], axis=0)

    def pad_idx(x):
        npad = _BPAD - x.shape[0]
        extra = (jnp.arange(npad * (x.size // x.shape[0]), dtype=jnp.int32)
                 % _N).reshape((npad,) + x.shape[1:])
        return jnp.concatenate([x.astype(jnp.int32), extra], axis=0)

    s32 = pad_idx(s)
    t32 = pad_idx(t)
    h32 = pad_idx(h_s)
    h_wjw = h32.reshape(_NW, _W, _H).transpose(0, 2, 1).reshape(-1)
    hstT = pad(h_s_times).T
    maskT = pad(h_s_mask).T
    edge2 = pad(edge_times_batch).reshape(1, _BPAD)

    s_rows, t_rows, h_rows, delta_g = _gather(
        emb_table, delta_table.reshape(_N), s32, t32, h_wjw)

    loss = _math(s_rows, t_rows, h_rows.reshape(_H, _BPAD, _D),
                 delta_g.reshape(1, _BPAD), edge2, hstT, maskT, sign)
    return loss[0, :_B]


# speedup vs baseline: 3.6064x; 1.2455x over previous
import functools

import jax
import jax.numpy as jnp
from jax import lax
from jax.experimental import pallas as pl
from jax.experimental.pallas import tpu as pltpu
from jax.experimental.pallas import tpu_sc as plsc

_N = 100000
_D = 128
_H = 5
_B = 100000

_NC, _NS = 2, 16
_NW = _NC * _NS
_W = 3200
_BPAD = _NW * _W
_C = 40
_NCH = _W // _C

_BLK = 512
_G = _BPAD // _BLK


def _gather_body(emb, dtab, sidx, tidx, hidx, s_out, t_out, h_out, d_out,
                 sidx_v, tidx_v, hidx_v, del_v,
                 rs0, rt0, rh0, rs1, rt1, rh1,
                 gsem0, gsem1, wsem0, wsem1):
    wid = lax.axis_index("s") * _NC + lax.axis_index("c")
    base = pl.multiple_of(wid * _W, 8)
    pltpu.sync_copy(sidx.at[pl.ds(base, _W)], sidx_v)
    pltpu.sync_copy(tidx.at[pl.ds(base, _W)], tidx_v)
    pltpu.sync_copy(hidx.at[pl.ds(base * _H, _W * _H)], hidx_v)

    sets = ((rs0, rt0, rh0, gsem0, wsem0), (rs1, rt1, rh1, gsem1, wsem1))

    def fire_gathers(g, bs):
        rs, rt, rh, gsem, _ = bs
        off = pl.multiple_of(g * _C, 8)
        pltpu.async_copy(emb.at[sidx_v.at[pl.ds(off, _C)]], rs, gsem)
        pltpu.async_copy(emb.at[tidx_v.at[pl.ds(off, _C)]], rt, gsem)
        pltpu.async_copy(dtab.at[sidx_v.at[pl.ds(off, _C)]],
                         del_v.at[pl.ds(off, _C)], gsem)
        for j in range(_H):
            pltpu.async_copy(emb.at[hidx_v.at[pl.ds(j * _W + off, _C)]],
                             rh.at[pl.ds(j * _C, _C)], gsem)

    def drain_gathers(bs):
        rs, rt, rh, gsem, _ = bs
        pltpu.make_async_copy(emb.at[sidx_v.at[pl.ds(0, _C)]], rs, gsem).wait()
        pltpu.make_async_copy(emb.at[sidx_v.at[pl.ds(0, _C)]], rt, gsem).wait()
        pltpu.make_async_copy(dtab.at[sidx_v.at[pl.ds(0, _C)]],
                              del_v.at[pl.ds(0, _C)], gsem).wait()
        for j in range(_H):
            pltpu.make_async_copy(emb.at[sidx_v.at[pl.ds(0, _C)]],
                                  rh.at[pl.ds(j * _C, _C)], gsem).wait()

    def fire_writes(g, bs):
        rs, rt, rh, _, wsem = bs
        gb = base + pl.multiple_of(g * _C, 8)
        pltpu.async_copy(rs, s_out.at[pl.ds(gb, _C)], wsem)
        pltpu.async_copy(rt, t_out.at[pl.ds(gb, _C)], wsem)
        for j in range(_H):
            pltpu.async_copy(rh.at[pl.ds(j * _C, _C)],
                             h_out.at[pl.ds(j * _BPAD + gb, _C)], wsem)

    def drain_writes(bs):
        rs, rt, rh, _, wsem = bs
        pltpu.make_async_copy(rs, s_out.at[pl.ds(base, _C)], wsem).wait()
        pltpu.make_async_copy(rt, t_out.at[pl.ds(base, _C)], wsem).wait()
        for j in range(_H):
            pltpu.make_async_copy(rh.at[pl.ds(j * _C, _C)],
                                  h_out.at[pl.ds(base, _C)], wsem).wait()

    fire_gathers(0, sets[0])

    def outer(i, carry):
        for b in range(2):
            g = 2 * i + b

            @pl.when(g + 1 < _NCH)
            def _fire_next():
                @pl.when(g >= 1)
                def _dw():
                    drain_writes(sets[1 - b])
                fire_gathers(g + 1, sets[1 - b])

            drain_gathers(sets[b])
            fire_writes(g, sets[b])
        return carry

    lax.fori_loop(0, _NCH // 2, outer, 0)
    drain_writes(sets[0])
    drain_writes(sets[1])
    pltpu.sync_copy(del_v, d_out.at[pl.ds(base, _W)])


def _math_body(s_ref, t_ref, h_ref, delta_ref, edge_ref, hst_ref, mask_ref,
               sign_ref, loss_ref):
    s_e = s_ref[...]
    t_e = t_ref[...]
    parts = [(s_e - t_e).astype(jnp.bfloat16)]
    for j in range(_H):
        parts.append((s_e - h_ref[j]).astype(jnp.bfloat16))
    p = jnp.concatenate([q * q for q in parts], axis=1)
    rows = lax.broadcasted_iota(jnp.int32, (6 * _D, 8), 0) // _D
    cols = lax.broadcasted_iota(jnp.int32, (6 * _D, 8), 1)
    w = (rows == cols).astype(jnp.bfloat16)
    r = lax.dot_general(p, w, (((1,), (0,)), ((), ())),
                        preferred_element_type=jnp.float32)
    rt = -r.T
    p_mu = rt[0:1]
    alpha = rt[1:1 + _H]
    m = jnp.max(alpha, axis=0, keepdims=True)
    es = jnp.exp(alpha - m)
    z_norm = jnp.sum(es, axis=0, keepdims=True)
    d_time = edge_ref[...] - hst_ref[...]
    wgt = (es / z_norm) * alpha * jnp.exp(-delta_ref[...] * d_time) * mask_ref[...]
    p_lambda = p_mu + jnp.sum(wgt, axis=0, keepdims=True)
    z = sign_ref[0] * p_lambda
    loss_ref[...] = jnp.maximum(-z, 0.0) + jnp.log(1.0 + jnp.exp(-jnp.abs(z)))


_sc_mesh = plsc.VectorSubcoreMesh(core_axis_name="c", subcore_axis_name="s")

_gather = pl.kernel(
    _gather_body,
    out_type=(
        jax.ShapeDtypeStruct((_BPAD, _D), jnp.float32),
        jax.ShapeDtypeStruct((_BPAD, _D), jnp.float32),
        jax.ShapeDtypeStruct((_H * _BPAD, _D), jnp.float32),
        jax.ShapeDtypeStruct((_BPAD,), jnp.float32),
    ),
    mesh=_sc_mesh,
    scratch_types=[
        pltpu.VMEM((_W,), jnp.int32),
        pltpu.VMEM((_W,), jnp.int32),
        pltpu.VMEM((_H * _W,), jnp.int32),
        pltpu.VMEM((_W,), jnp.float32),
        pltpu.VMEM((_C, _D), jnp.float32),
        pltpu.VMEM((_C, _D), jnp.float32),
        pltpu.VMEM((_H * _C, _D), jnp.float32),
        pltpu.VMEM((_C, _D), jnp.float32),
        pltpu.VMEM((_C, _D), jnp.float32),
        pltpu.VMEM((_H * _C, _D), jnp.float32),
        pltpu.SemaphoreType.DMA,
        pltpu.SemaphoreType.DMA,
        pltpu.SemaphoreType.DMA,
        pltpu.SemaphoreType.DMA,
    ],
)

_math = pl.pallas_call(
    _math_body,
    grid=(_G,),
    in_specs=[
        pl.BlockSpec((_BLK, _D), lambda i: (i, 0)),
        pl.BlockSpec((_BLK, _D), lambda i: (i, 0)),
        pl.BlockSpec((_H, _BLK, _D), lambda i: (0, i, 0)),
        pl.BlockSpec((1, _BLK), lambda i: (0, i)),
        pl.BlockSpec((1, _BLK), lambda i: (0, i)),
        pl.BlockSpec((_H, _BLK), lambda i: (0, i)),
        pl.BlockSpec((_H, _BLK), lambda i: (0, i)),
        pl.BlockSpec(memory_space=pltpu.SMEM),
    ],
    out_specs=pl.BlockSpec((1, _BLK), lambda i: (0, i)),
    out_shape=jax.ShapeDtypeStruct((1, _BPAD), jnp.float32),
)


def kernel(sign, s, t, edge_times_batch, h_s, h_s_times, h_s_mask, emb_table,
           delta_table):
    def pad(x):
        return jnp.concatenate(
            [x, jnp.zeros((_BPAD - _B,) + x.shape[1:], x.dtype)], axis=0)

    def pad_idx(x):
        npad = _BPAD - x.shape[0]
        extra = (jnp.arange(npad * (x.size // x.shape[0]), dtype=jnp.int32)
                 % _N).reshape((npad,) + x.shape[1:])
        return jnp.concatenate([x.astype(jnp.int32), extra], axis=0)

    s32 = pad_idx(s)
    t32 = pad_idx(t)
    h32 = pad_idx(h_s)
    h_wjw = h32.reshape(_NW, _W, _H).transpose(0, 2, 1).reshape(-1)
    hstT = pad(h_s_times).T
    maskT = pad(h_s_mask).T
    edge2 = pad(edge_times_batch).reshape(1, _BPAD)

    s_rows, t_rows, h_rows, delta_g = _gather(
        emb_table, delta_table.reshape(_N), s32, t32, h_wjw)

    loss = _math(s_rows, t_rows, h_rows.reshape(_H, _BPAD, _D),
                 delta_g.reshape(1, _BPAD), edge2, hstT, maskT, sign)
    return loss[0, :_B]
